# baseline (speedup 1.0000x reference)
import functools
import math

import jax
import jax.numpy as jnp
from jax import lax
from jax.experimental import pallas as pl
from jax.experimental.pallas import tpu as pltpu
from jax.experimental.pallas import tpu_sc as plsc

D_MODEL = 64
SCALE = math.sqrt(D_MODEL)
VOCAB_ROWS = 1000000
TCOLS = 7813
VPAD = TCOLS * 128

NC = 2
NS = 16
NW = NC * NS

NB = 4096
NJ = 200
IB = NB // NW

_mesh = plsc.VectorSubcoreMesh(core_axis_name="c", subcore_axis_name="s")


_S_STEPS = 246


@functools.partial(
    pl.kernel,
    out_type=jax.ShapeDtypeStruct((VPAD // 2, 128), jnp.float32),
    mesh=_mesh,
    scratch_types=(
        [pltpu.VMEM((D_MODEL, 128), jnp.float32)] * 6
        + [pltpu.VMEM((D_MODEL, 128), jnp.float32)] * 6
        + [pltpu.SemaphoreType.DMA] * 12
    ),
    compiler_params=pltpu.CompilerParams(
        use_tc_tiling_on_sc=True, needs_layout_passes=False
    ),
)
def _pack(tT_hbm, tp_hbm,
          gbuf0, gbuf1, gbuf2, gbuf3, gbuf4, gbuf5,
          obuf0, obuf1, obuf2, obuf3, obuf4, obuf5,
          gsem0, gsem1, gsem2, gsem3, gsem4, gsem5,
          ssem0, ssem1, ssem2, ssem3, ssem4, ssem5):
    wid = lax.axis_index("s") * NC + lax.axis_index("c")

    gbufs = (gbuf0, gbuf1, gbuf2, gbuf3, gbuf4, gbuf5)
    obufs = (obuf0, obuf1, obuf2, obuf3, obuf4, obuf5)
    gsems = (gsem0, gsem1, gsem2, gsem3, gsem4, gsem5)
    ssems = (ssem0, ssem1, ssem2, ssem3, ssem4, ssem5)

    iota16 = lax.iota(jnp.int32, 16)
    rows4 = [iota16 + (u * 16) for u in range(4)]
    rots = [(iota16 + d) & 15 for d in range(16)]

    def fire_gather(rt, b):
        for ct in range(8):
            pltpu.async_copy(
                tT_hbm.at[pl.ds(ct * 8, 8), pl.ds(rt * 128, 128)],
                gbufs[b].at[pl.ds(ct * 8, 8)],
                gsems[b],
            )

    def wait_gather(rt, b):
        for ct in range(8):
            pltpu.make_async_copy(
                tT_hbm.at[pl.ds(ct * 8, 8), pl.ds(rt * 128, 128)],
                gbufs[b].at[pl.ds(ct * 8, 8)],
                gsems[b],
            ).wait()

    def process(b):
        gbuf = gbufs[b]
        obuf = obufs[b]

        @plsc.parallel_loop(0, 16, step=1, unroll=2)
        def _(q):
            rowv = iota16 + ((q >> 2) << 4)
            qb = jnp.full((16,), 0, jnp.int32) + ((q & 3) << 4)
            for p in range(2):
                for d in range(16):
                    qd = qb + rots[d]
                    v = plsc.load_gather(gbuf, [rowv, qd * 2 + p])
                    plsc.store_scatter(obuf, [qd, rowv + (64 * p)], v)

    def fire_store(rt, b):
        pltpu.async_copy(obufs[b], tp_hbm.at[pl.ds(rt * 64, 64)], ssems[b])

    def wait_store(rt, b):
        pltpu.make_async_copy(
            obufs[b], tp_hbm.at[pl.ds(rt * 64, 64)], ssems[b]
        ).wait()

    for b in range(6):
        fire_gather(wid + b * NW, b)

    @pl.loop(0, _S_STEPS, step=6)
    def _(g):
        for b in range(6):
            s = g + b
            rt = wid + s * NW

            @pl.when(rt < TCOLS)
            def _():
                wait_gather(rt, b)

                @pl.when(s >= 6)
                def _():
                    wait_store(rt - 6 * NW, b)

                process(b)

                @pl.when(rt + 6 * NW < TCOLS)
                def _():
                    fire_gather(rt + 6 * NW, b)

                fire_store(rt, b)

    for s in range(_S_STEPS - 12, _S_STEPS):
        b = s % 6
        rt_s = wid + s * NW
        if s + 6 >= _S_STEPS:
            cond = rt_s < TCOLS
        else:
            cond = (rt_s < TCOLS) & (rt_s + 6 * NW >= TCOLS)

        @pl.when(cond)
        def _():
            wait_store(rt_s, b)


@functools.partial(
    pl.kernel,
    out_type=jax.ShapeDtypeStruct((NJ, 8, NW, 8, 128), jnp.float32),
    mesh=_mesh,
    scratch_types=(
        [pltpu.VMEM((NJ, IB), jnp.int32)]
        + [pltpu.VMEM((IB, D_MODEL), jnp.float32)] * 8
        + [pltpu.VMEM((D_MODEL, 128), jnp.float32)] * 2
        + [pltpu.SemaphoreType.DMA] * 10
    ),
    compiler_params=pltpu.CompilerParams(
        use_tc_tiling_on_sc=False, needs_layout_passes=False
    ),
)
def _emb5(x_hbm, table_hbm, out_hbm, idx_all,
          gbuf0, gbuf1, gbuf2, gbuf3, gbuf4, gbuf5, gbuf6, gbuf7,
          obuf0, obuf1,
          gsem0, gsem1, gsem2, gsem3, gsem4, gsem5, gsem6, gsem7,
          ssem0, ssem1):
    wid = lax.axis_index("s") * NC + lax.axis_index("c")

    gbufs = (gbuf0, gbuf1, gbuf2, gbuf3, gbuf4, gbuf5, gbuf6, gbuf7)
    obufs = (obuf0, obuf1)
    gsems = (gsem0, gsem1, gsem2, gsem3, gsem4, gsem5, gsem6, gsem7)
    ssems = (ssem0, ssem1)

    pltpu.sync_copy(x_hbm.at[wid], idx_all)

    iota16 = lax.iota(jnp.int32, 16)
    rows = [iota16 + (t * 16) for t in range(8)]
    rots = [(iota16 + d) & 15 for d in range(16)]

    def fire_gather(j, b):
        pltpu.async_copy(table_hbm.at[idx_all.at[j]], gbufs[b], gsems[b])

    def wait_gather(j, b):
        pltpu.make_async_copy(
            table_hbm.at[idx_all.at[j]], gbufs[b], gsems[b]
        ).wait()

    def process(j, b):
        gbuf = gbufs[b]
        obuf = obufs[b % 2]

        @plsc.parallel_loop(0, 32, step=1, unroll=2)
        def _(q):
            rowv = iota16 + ((q >> 2) << 4)
            kb = jnp.full((16,), 0, jnp.int32) + ((q & 3) << 4)
            for d in range(16):
                kd = kb + rots[d]
                v = plsc.load_gather(gbuf, [rowv, kd])
                plsc.store_scatter(obuf, [kd, rowv], v * SCALE)

    def fire_store(j, b):
        for kt in range(8):
            pltpu.async_copy(
                obufs[b].at[pl.ds(kt * 8, 8)],
                out_hbm.at[j, kt, wid],
                ssems[b],
            )

    def wait_store(j, b):
        for kt in range(8):
            pltpu.make_async_copy(
                obufs[b].at[pl.ds(kt * 8, 8)],
                out_hbm.at[j, kt, wid],
                ssems[b],
            ).wait()

    for b in range(8):
        fire_gather(b, b)

    @pl.loop(0, NJ, step=8)
    def _(g):
        for b in range(8):
            j = g + b
            wait_gather(j, b)

            @pl.when(j >= 2)
            def _():
                wait_store(j - 2, b % 2)

            process(j, b)

            @pl.when(j + 8 < NJ)
            def _():
                fire_gather(j + 8, b)

            fire_store(j, b % 2)

    for b in range(2):
        wait_store(NJ - 2 + b, b)


def kernel(x, table):
    xi = x.astype(jnp.int32)
    xprep = xi.reshape(NW, IB, NJ).transpose(0, 2, 1)
    tpairs = _pack(jnp.transpose(table))
    tdense = tpairs.reshape(VPAD, D_MODEL)
    y5 = _emb5(xprep, tdense)
    return y5.transpose(2, 4, 0, 1, 3).reshape(NB, NJ, D_MODEL)


# speedup vs baseline: 1.5260x; 1.5260x over previous
import functools
import math

import jax
import jax.numpy as jnp
from jax import lax
from jax.experimental import pallas as pl
from jax.experimental.pallas import tpu as pltpu
from jax.experimental.pallas import tpu_sc as plsc

D_MODEL = 64
SCALE = math.sqrt(D_MODEL)
VOCAB_ROWS = 1000000
TCOLS = 7813
VPAD = TCOLS * 128

NC = 2
NS = 16
NW = NC * NS

NB = 4096
NJ = 200
IB = NB // NW

_mesh = plsc.VectorSubcoreMesh(core_axis_name="c", subcore_axis_name="s")


_S_STEPS = 246


@functools.partial(
    pl.kernel,
    out_type=jax.ShapeDtypeStruct((VPAD // 2, 128), jnp.float32),
    mesh=_mesh,
    scratch_types=(
        [pltpu.VMEM((D_MODEL, 128), jnp.float32)] * 4
        + [pltpu.VMEM((D_MODEL, 128), jnp.float32)] * 4
        + [pltpu.SemaphoreType.DMA] * 8
    ),
    compiler_params=pltpu.CompilerParams(
        use_tc_tiling_on_sc=True, needs_layout_passes=False
    ),
)
def _pack(tT_hbm, tp_hbm,
          gbuf0, gbuf1, gbuf2, gbuf3, obuf0, obuf1, obuf2, obuf3,
          gsem0, gsem1, gsem2, gsem3, ssem0, ssem1, ssem2, ssem3):
    wid = lax.axis_index("s") * NC + lax.axis_index("c")

    gbufs = (gbuf0, gbuf1, gbuf2, gbuf3)
    obufs = (obuf0, obuf1, obuf2, obuf3)
    gsems = (gsem0, gsem1, gsem2, gsem3)
    ssems = (ssem0, ssem1, ssem2, ssem3)

    iota16 = lax.iota(jnp.int32, 16)
    rows4 = [iota16 + (u * 16) for u in range(4)]
    rots = [(iota16 + d) & 15 for d in range(16)]

    def fire_gather(rt, b):
        for ct in range(8):
            pltpu.async_copy(
                tT_hbm.at[pl.ds(ct * 8, 8), pl.ds(rt * 128, 128)],
                gbufs[b].at[pl.ds(ct * 8, 8)],
                gsems[b],
            )

    def wait_gather(rt, b):
        for ct in range(8):
            pltpu.make_async_copy(
                tT_hbm.at[pl.ds(ct * 8, 8), pl.ds(rt * 128, 128)],
                gbufs[b].at[pl.ds(ct * 8, 8)],
                gsems[b],
            ).wait()

    def process(b):
        gbuf = gbufs[b]
        obuf = obufs[b]

        for I in range(4):
            rowv = rows4[I]

            @plsc.parallel_loop(0, 4, step=1, unroll=2)
            def _(Q):
                qb = jnp.full((16,), 0, jnp.int32) + (Q * 16)
                for p in range(2):
                    for d in range(16):
                        qd = qb + rots[d]
                        v = plsc.load_gather(gbuf, [rowv, qd * 2 + p])
                        plsc.store_scatter(obuf, [qd, rowv + (64 * p)], v)

    def fire_store(rt, b):
        pltpu.async_copy(obufs[b], tp_hbm.at[pl.ds(rt * 64, 64)], ssems[b])

    def wait_store(rt, b):
        pltpu.make_async_copy(
            obufs[b], tp_hbm.at[pl.ds(rt * 64, 64)], ssems[b]
        ).wait()

    for b in range(4):
        fire_gather(wid + b * NW, b)

    @pl.loop(0, _S_STEPS, step=4)
    def _(g):
        for b in range(4):
            s = g + b
            rt = wid + s * NW

            @pl.when(rt < TCOLS)
            def _():
                wait_gather(rt, b)

                @pl.when(s >= 4)
                def _():
                    wait_store(rt - 4 * NW, b)

                process(b)

                @pl.when(rt + 4 * NW < TCOLS)
                def _():
                    fire_gather(rt + 4 * NW, b)

                fire_store(rt, b)

    for s in range(_S_STEPS - 8, _S_STEPS):
        b = s % 4
        rt_s = wid + s * NW
        if s + 4 >= _S_STEPS:
            cond = rt_s < TCOLS
        else:
            cond = (rt_s < TCOLS) & (rt_s + 4 * NW >= TCOLS)

        @pl.when(cond)
        def _():
            wait_store(rt_s, b)


@functools.partial(
    pl.kernel,
    out_type=jax.ShapeDtypeStruct((NJ, 8, NW, 8, 128), jnp.float32),
    mesh=_mesh,
    scratch_types=(
        [pltpu.VMEM((NJ, IB), jnp.int32)]
        + [pltpu.VMEM((IB, D_MODEL), jnp.float32)] * 4
        + [pltpu.VMEM((D_MODEL, 128), jnp.float32)] * 4
        + [pltpu.SemaphoreType.DMA] * 8
    ),
    compiler_params=pltpu.CompilerParams(
        use_tc_tiling_on_sc=False, needs_layout_passes=False
    ),
)
def _emb5(x_hbm, table_hbm, out_hbm, idx_all,
          gbuf0, gbuf1, gbuf2, gbuf3, obuf0, obuf1, obuf2, obuf3,
          gsem0, gsem1, gsem2, gsem3, ssem0, ssem1, ssem2, ssem3):
    wid = lax.axis_index("s") * NC + lax.axis_index("c")

    gbufs = (gbuf0, gbuf1, gbuf2, gbuf3)
    obufs = (obuf0, obuf1, obuf2, obuf3)
    gsems = (gsem0, gsem1, gsem2, gsem3)
    ssems = (ssem0, ssem1, ssem2, ssem3)

    pltpu.sync_copy(x_hbm.at[wid], idx_all)

    iota16 = lax.iota(jnp.int32, 16)
    rows = [iota16 + (t * 16) for t in range(8)]
    rots = [(iota16 + d) & 15 for d in range(16)]

    def fire_gather(j, b):
        pltpu.async_copy(table_hbm.at[idx_all.at[j]], gbufs[b], gsems[b])

    def wait_gather(j, b):
        pltpu.make_async_copy(
            table_hbm.at[idx_all.at[j]], gbufs[b], gsems[b]
        ).wait()

    def process(j, b):
        gbuf = gbufs[b]
        obuf = obufs[b]

        for I in range(8):
            rowv = rows[I]

            @plsc.parallel_loop(0, 4, step=1, unroll=2)
            def _(K):
                kb = jnp.full((16,), 0, jnp.int32) + (K * 16)
                for d in range(16):
                    kd = kb + rots[d]
                    v = plsc.load_gather(gbuf, [rowv, kd])
                    plsc.store_scatter(obuf, [kd, rowv], v * SCALE)

    def fire_store(j, b):
        for kt in range(8):
            pltpu.async_copy(
                obufs[b].at[pl.ds(kt * 8, 8)],
                out_hbm.at[j, kt, wid],
                ssems[b],
            )

    def wait_store(j, b):
        for kt in range(8):
            pltpu.make_async_copy(
                obufs[b].at[pl.ds(kt * 8, 8)],
                out_hbm.at[j, kt, wid],
                ssems[b],
            ).wait()

    for b in range(4):
        fire_gather(b, b)

    @pl.loop(0, NJ, step=4)
    def _(g):
        for b in range(4):
            j = g + b
            wait_gather(j, b)

            @pl.when(j >= 4)
            def _():
                wait_store(j - 4, b)

            process(j, b)

            @pl.when(j + 4 < NJ)
            def _():
                fire_gather(j + 4, b)

            fire_store(j, b)

    for b in range(4):
        wait_store(NJ - 4 + b, b)


def kernel(x, table):
    xi = x.astype(jnp.int32)
    xprep = xi.reshape(NW, IB, NJ).transpose(0, 2, 1)
    tpairs = _pack(jnp.transpose(table))
    tdense = tpairs.reshape(VPAD, D_MODEL)
    y5 = _emb5(xprep, tdense)
    return y5.transpose(2, 4, 0, 1, 3).reshape(NB, NJ, D_MODEL)
